# baseline (speedup 1.0000x reference)
import functools

import jax
import jax.numpy as jnp
from jax import lax
from jax.experimental import pallas as pl
from jax.experimental.pallas import tpu as pltpu
from jax.experimental.pallas import tpu_sc as plsc

N_ROWS = 1000000
EMB_DIM = 16
HIDDEN_DIM = 128
BATCH = 16384

WIDE = 128
PACK = WIDE // EMB_DIM

NC, NS = 2, 16
NW = NC * NS
BPW = BATCH // NW
CHUNK = 128
NCH = BPW // CHUNK

MLP_BLK = 2048
TP_BLK = 16384


def _transpose_pack_body(x_ref, o_ref):
    xt = jnp.swapaxes(x_ref[...], 0, 1)
    x3 = xt.reshape(TP_BLK // PACK, PACK, EMB_DIM)
    o_ref[...] = jnp.concatenate([x3[:, j, :] for j in range(PACK)], axis=1)


def _transpose_pack_call(tT):
    return pl.pallas_call(
        _transpose_pack_body,
        grid=(pl.cdiv(N_ROWS, TP_BLK),),
        in_specs=[pl.BlockSpec((EMB_DIM, TP_BLK), lambda j: (0, j))],
        out_specs=pl.BlockSpec((TP_BLK // PACK, WIDE), lambda j: (j, 0)),
        out_shape=jax.ShapeDtypeStruct((N_ROWS // PACK, WIDE), jnp.float32),
    )(tT)


def _gather_body(user_hbm, item_hbm, u2_hbm, i2_hbm, ue_hbm, ie_hbm,
                 idx_u, idx_i, rows, sem):
    wid = lax.axis_index("s") * NC + lax.axis_index("c")
    base = wid * BPW
    pltpu.sync_copy(u2_hbm.at[pl.ds(wid * NCH, NCH)], idx_u)
    pltpu.sync_copy(i2_hbm.at[pl.ds(wid * NCH, NCH)], idx_i)
    copies = []
    for j in range(NCH):
        copies.append(pltpu.async_copy(
            user_hbm.at[idx_u.at[j]], rows.at[pl.ds(j * CHUNK, CHUNK)], sem))
    for c in copies:
        c.wait()
    pltpu.sync_copy(rows, ue_hbm.at[pl.ds(base, BPW)])
    copies = []
    for j in range(NCH):
        copies.append(pltpu.async_copy(
            item_hbm.at[idx_i.at[j]], rows.at[pl.ds(j * CHUNK, CHUNK)], sem))
    for c in copies:
        c.wait()
    pltpu.sync_copy(rows, ie_hbm.at[pl.ds(base, BPW)])


def _mlp_body(su_ref, si_ref, uw_ref, iw_ref, w1u_ref, w1i_ref, b1_ref,
              w2_ref, b2_ref, out_ref):
    grp = lax.broadcasted_iota(jnp.int32, (MLP_BLK, WIDE), 1) // EMB_DIM
    xu = jnp.where(grp == su_ref[...], uw_ref[...], 0.0)
    xi = jnp.where(grp == si_ref[...], iw_ref[...], 0.0)
    h = (jnp.dot(xu, w1u_ref[...], preferred_element_type=jnp.float32)
         + jnp.dot(xi, w1i_ref[...], preferred_element_type=jnp.float32)
         + b1_ref[...])
    h = jnp.maximum(h, 0.0)
    out_ref[...] = jnp.sum(h * w2_ref[...], axis=1) + b2_ref[0, 0]


def _gather_call(uw, iw, u2, i2):
    return pl.kernel(
        _gather_body,
        mesh=plsc.VectorSubcoreMesh(core_axis_name="c", subcore_axis_name="s"),
        compiler_params=pltpu.CompilerParams(use_tc_tiling_on_sc=False),
        out_type=[jax.ShapeDtypeStruct((BATCH, WIDE), jnp.float32),
                  jax.ShapeDtypeStruct((BATCH, WIDE), jnp.float32)],
        scratch_types=[
            pltpu.VMEM((NCH, CHUNK), jnp.int32),
            pltpu.VMEM((NCH, CHUNK), jnp.int32),
            pltpu.VMEM((BPW, WIDE), jnp.float32),
            pltpu.SemaphoreType.DMA,
        ],
    )(uw, iw, u2, i2)


def _mlp_call(su, si, uew, iew, w1u_rep, w1i_rep, b1r, w2r, b2r):
    return pl.pallas_call(
        _mlp_body,
        grid=(BATCH // MLP_BLK,),
        in_specs=[
            pl.BlockSpec((MLP_BLK, 1), lambda j: (j, 0)),
            pl.BlockSpec((MLP_BLK, 1), lambda j: (j, 0)),
            pl.BlockSpec((MLP_BLK, WIDE), lambda j: (j, 0)),
            pl.BlockSpec((MLP_BLK, WIDE), lambda j: (j, 0)),
            pl.BlockSpec((WIDE, HIDDEN_DIM), lambda j: (0, 0)),
            pl.BlockSpec((WIDE, HIDDEN_DIM), lambda j: (0, 0)),
            pl.BlockSpec((1, HIDDEN_DIM), lambda j: (0, 0)),
            pl.BlockSpec((1, HIDDEN_DIM), lambda j: (0, 0)),
            pl.BlockSpec((1, 1), lambda j: (0, 0)),
        ],
        out_specs=pl.BlockSpec((MLP_BLK,), lambda j: (j,)),
        out_shape=jax.ShapeDtypeStruct((BATCH,), jnp.float32),
    )(su, si, uew, iew, w1u_rep, w1i_rep, b1r, w2r, b2r)


def kernel(u, i, user_emb, item_emb, W1, b1, W2, b2):
    uw = _transpose_pack_call(user_emb.T)
    iw = _transpose_pack_call(item_emb.T)
    u32 = u.astype(jnp.int32)
    i32 = i.astype(jnp.int32)
    u2 = (u32 >> 3).reshape(BATCH // CHUNK, CHUNK)
    i2 = (i32 >> 3).reshape(BATCH // CHUNK, CHUNK)
    ue_w, ie_w = _gather_call(uw, iw, u2, i2)
    su = (u32 & 7).reshape(BATCH, 1)
    si = (i32 & 7).reshape(BATCH, 1)
    w1u_rep = jnp.tile(W1[:EMB_DIM], (PACK, 1))
    w1i_rep = jnp.tile(W1[EMB_DIM:], (PACK, 1))
    b1r = b1.reshape(1, HIDDEN_DIM)
    w2r = W2.reshape(1, HIDDEN_DIM)
    b2r = b2.reshape(1, 1)
    return _mlp_call(su, si, ue_w, ie_w, w1u_rep, w1i_rep, b1r, w2r, b2r)


# speedup vs baseline: 1.0048x; 1.0048x over previous
import functools

import jax
import jax.numpy as jnp
from jax import lax
from jax.experimental import pallas as pl
from jax.experimental.pallas import tpu as pltpu
from jax.experimental.pallas import tpu_sc as plsc

N_ROWS = 1000000
EMB_DIM = 16
HIDDEN_DIM = 128
BATCH = 16384

WIDE = 128
PACK = WIDE // EMB_DIM

NC, NS = 2, 16
NW = NC * NS
BPW = BATCH // NW
CHUNK = 128
NCH = BPW // CHUNK

MLP_BLK = 2048
TP_BLK = 8192


def _transpose_pack_body(x_ref, o_ref):
    xt = jnp.swapaxes(x_ref[...], 0, 1)
    x3 = xt.reshape(TP_BLK // PACK, PACK, EMB_DIM)
    o_ref[...] = jnp.concatenate([x3[:, j, :] for j in range(PACK)], axis=1)


def _transpose_pack_call(tT):
    return pl.pallas_call(
        _transpose_pack_body,
        grid=(pl.cdiv(N_ROWS, TP_BLK),),
        in_specs=[pl.BlockSpec((EMB_DIM, TP_BLK), lambda j: (0, j))],
        out_specs=pl.BlockSpec((TP_BLK // PACK, WIDE), lambda j: (j, 0)),
        out_shape=jax.ShapeDtypeStruct((N_ROWS // PACK, WIDE), jnp.float32),
    )(tT)


def _gather_body(user_hbm, item_hbm, u2_hbm, i2_hbm, ue_hbm, ie_hbm,
                 idx_u, idx_i, rows, sem):
    wid = lax.axis_index("s") * NC + lax.axis_index("c")
    base = wid * BPW
    pltpu.sync_copy(u2_hbm.at[pl.ds(wid * NCH, NCH)], idx_u)
    pltpu.sync_copy(i2_hbm.at[pl.ds(wid * NCH, NCH)], idx_i)
    copies = []
    for j in range(NCH):
        copies.append(pltpu.async_copy(
            user_hbm.at[idx_u.at[j]], rows.at[pl.ds(j * CHUNK, CHUNK)], sem))
    for c in copies:
        c.wait()
    pltpu.sync_copy(rows, ue_hbm.at[pl.ds(base, BPW)])
    copies = []
    for j in range(NCH):
        copies.append(pltpu.async_copy(
            item_hbm.at[idx_i.at[j]], rows.at[pl.ds(j * CHUNK, CHUNK)], sem))
    for c in copies:
        c.wait()
    pltpu.sync_copy(rows, ie_hbm.at[pl.ds(base, BPW)])


def _mlp_body(su_ref, si_ref, uw_ref, iw_ref, w1u_ref, w1i_ref, b1_ref,
              w2_ref, b2_ref, out_ref):
    grp = lax.broadcasted_iota(jnp.int32, (MLP_BLK, WIDE), 1) // EMB_DIM
    xu = jnp.where(grp == su_ref[...], uw_ref[...], 0.0)
    xi = jnp.where(grp == si_ref[...], iw_ref[...], 0.0)
    h = (jnp.dot(xu, w1u_ref[...], preferred_element_type=jnp.float32)
         + jnp.dot(xi, w1i_ref[...], preferred_element_type=jnp.float32)
         + b1_ref[...])
    h = jnp.maximum(h, 0.0)
    out_ref[...] = jnp.sum(h * w2_ref[...], axis=1) + b2_ref[0, 0]


def _gather_call(uw, iw, u2, i2):
    return pl.kernel(
        _gather_body,
        mesh=plsc.VectorSubcoreMesh(core_axis_name="c", subcore_axis_name="s"),
        compiler_params=pltpu.CompilerParams(use_tc_tiling_on_sc=False),
        out_type=[jax.ShapeDtypeStruct((BATCH, WIDE), jnp.float32),
                  jax.ShapeDtypeStruct((BATCH, WIDE), jnp.float32)],
        scratch_types=[
            pltpu.VMEM((NCH, CHUNK), jnp.int32),
            pltpu.VMEM((NCH, CHUNK), jnp.int32),
            pltpu.VMEM((BPW, WIDE), jnp.float32),
            pltpu.SemaphoreType.DMA,
        ],
    )(uw, iw, u2, i2)


def _mlp_call(su, si, uew, iew, w1u_rep, w1i_rep, b1r, w2r, b2r):
    return pl.pallas_call(
        _mlp_body,
        grid=(BATCH // MLP_BLK,),
        in_specs=[
            pl.BlockSpec((MLP_BLK, 1), lambda j: (j, 0)),
            pl.BlockSpec((MLP_BLK, 1), lambda j: (j, 0)),
            pl.BlockSpec((MLP_BLK, WIDE), lambda j: (j, 0)),
            pl.BlockSpec((MLP_BLK, WIDE), lambda j: (j, 0)),
            pl.BlockSpec((WIDE, HIDDEN_DIM), lambda j: (0, 0)),
            pl.BlockSpec((WIDE, HIDDEN_DIM), lambda j: (0, 0)),
            pl.BlockSpec((1, HIDDEN_DIM), lambda j: (0, 0)),
            pl.BlockSpec((1, HIDDEN_DIM), lambda j: (0, 0)),
            pl.BlockSpec((1, 1), lambda j: (0, 0)),
        ],
        out_specs=pl.BlockSpec((MLP_BLK,), lambda j: (j,)),
        out_shape=jax.ShapeDtypeStruct((BATCH,), jnp.float32),
    )(su, si, uew, iew, w1u_rep, w1i_rep, b1r, w2r, b2r)


def kernel(u, i, user_emb, item_emb, W1, b1, W2, b2):
    uw = _transpose_pack_call(user_emb.T)
    iw = _transpose_pack_call(item_emb.T)
    u32 = u.astype(jnp.int32)
    i32 = i.astype(jnp.int32)
    u2 = (u32 >> 3).reshape(BATCH // CHUNK, CHUNK)
    i2 = (i32 >> 3).reshape(BATCH // CHUNK, CHUNK)
    ue_w, ie_w = _gather_call(uw, iw, u2, i2)
    su = (u32 & 7).reshape(BATCH, 1)
    si = (i32 & 7).reshape(BATCH, 1)
    w1u_rep = jnp.tile(W1[:EMB_DIM], (PACK, 1))
    w1i_rep = jnp.tile(W1[EMB_DIM:], (PACK, 1))
    b1r = b1.reshape(1, HIDDEN_DIM)
    w2r = W2.reshape(1, HIDDEN_DIM)
    b2r = b2.reshape(1, 1)
    return _mlp_call(su, si, ue_w, ie_w, w1u_rep, w1i_rep, b1r, w2r, b2r)
